# baseline (speedup 1.0000x reference)
import jax
import jax.numpy as jnp
from jax.experimental import pallas as pl
from jax.experimental.pallas import tpu as pltpu

_BS, _N, _D, _K = 8, 4096, 64, 512
_MAX_ITER = 8
_SEED = 123
_NCHUNK = 2048


def _init_centers_like_ref(x):
    key = jax.random.key(_SEED)
    keys = jax.random.split(key, _BS)

    def pick(xi, k):
        idx = jax.random.choice(k, _N, shape=(_K,), replace=False)
        return jnp.take(xi, idx, axis=0)

    return jax.vmap(pick)(x, keys)


def _kmeans_body(x_ref, c0_ref, labels_ref, centers_ref, inertia_ref):
    x = x_ref[0]
    x2 = jnp.sum(x * x, axis=1, keepdims=True)
    x2r = x2.T
    iota_sub = jax.lax.broadcasted_iota(
        jnp.int32, (_K, _NCHUNK), 0).astype(jnp.float32)
    ones_col = jnp.ones((_NCHUNK, 1), jnp.float32)

    def assign_chunk(s, centers2, c2):
        xw = x[s * _NCHUNK:(s + 1) * _NCHUNK]
        mm = jax.lax.dot_general(
            centers2, xw, (((1,), (1,)), ((), ())),
            preferred_element_type=jnp.float32)
        t = (x2r[:, s * _NCHUNK:(s + 1) * _NCHUNK] + c2) - mm
        tmin = jnp.min(t, axis=0, keepdims=True)
        d2min = jnp.maximum(tmin, 0.0)
        mask = t <= d2min
        wi = jnp.where(mask, iota_sub, jnp.float32(_K))
        labels = jnp.min(wi, axis=0, keepdims=True)
        onehot_t = (wi == labels).astype(jnp.float32)
        return xw, d2min, labels, onehot_t

    def body(_, centers):
        c2 = jnp.sum(centers * centers, axis=1, keepdims=True)
        centers2 = centers + centers
        sums_aug = jnp.zeros((_K, _D + 1), jnp.float32)
        for s in range(_N // _NCHUNK):
            xw, _, _, onehot_t = assign_chunk(s, centers2, c2)
            xw_aug = jnp.concatenate([xw, ones_col], axis=1)
            sums_aug = sums_aug + jnp.dot(
                onehot_t, xw_aug, preferred_element_type=jnp.float32)
        sums = sums_aug[:, :_D]
        counts = sums_aug[:, _D:]
        new_centers = sums / jnp.maximum(counts, 1.0)
        return jnp.where(counts > 0, new_centers, centers)

    centers = jax.lax.fori_loop(0, _MAX_ITER, body, c0_ref[0])

    c2 = jnp.sum(centers * centers, axis=1, keepdims=True)
    centers2 = centers + centers
    acc = jnp.zeros((), jnp.float32)
    for s in range(_N // _NCHUNK):
        _, d2min, labels, _ = assign_chunk(s, centers2, c2)
        labels_ref[0, 0, pl.ds(s * _NCHUNK, _NCHUNK)] = (
            labels.astype(jnp.int32).reshape(_NCHUNK))
        acc = acc + jnp.sum(d2min)
    centers_ref[0] = centers
    inertia_ref[...] = acc.reshape(1, 1, 1)


def kernel(x):
    c0 = _init_centers_like_ref(x)
    labels3, centers, inertia2 = pl.pallas_call(
        _kmeans_body,
        grid=(_BS,),
        in_specs=[
            pl.BlockSpec((1, _N, _D), lambda i: (i, 0, 0)),
            pl.BlockSpec((1, _K, _D), lambda i: (i, 0, 0)),
        ],
        out_specs=[
            pl.BlockSpec((1, 1, _N), lambda i: (i, 0, 0)),
            pl.BlockSpec((1, _K, _D), lambda i: (i, 0, 0)),
            pl.BlockSpec((1, 1, 1), lambda i: (i, 0, 0)),
        ],
        out_shape=[
            jax.ShapeDtypeStruct((_BS, 1, _N), jnp.int32),
            jax.ShapeDtypeStruct((_BS, _K, _D), jnp.float32),
            jax.ShapeDtypeStruct((_BS, 1, 1), jnp.float32),
        ],
        compiler_params=pltpu.CompilerParams(
            dimension_semantics=("parallel",)),
    )(x, c0)
    return labels3.reshape(_BS, _N), centers, inertia2.reshape(_BS)


# speedup vs baseline: 1.1162x; 1.1162x over previous
import jax
import jax.numpy as jnp
from jax.experimental import pallas as pl
from jax.experimental.pallas import tpu as pltpu

_BS, _N, _D, _K = 8, 4096, 64, 512
_MAX_ITER = 8
_SEED = 123
_NCHUNK = 2048
_BPG = 2


def _init_centers_like_ref(x):
    key = jax.random.key(_SEED)
    keys = jax.random.split(key, _BS)

    def pick(xi, k):
        idx = jax.random.choice(k, _N, shape=(_K,), replace=False)
        return jnp.take(xi, idx, axis=0)

    return jax.vmap(pick)(x, keys)


def _kmeans_body(x_ref, c0_ref, labels_ref, centers_ref, inertia_ref):
    iota_sub = jax.lax.broadcasted_iota(
        jnp.int32, (_K, _NCHUNK), 0).astype(jnp.float32)
    ones_col = jnp.ones((_NCHUNK, 1), jnp.float32)

    xs = [x_ref[b] for b in range(_BPG)]
    x2rs = [jnp.sum(xb * xb, axis=1, keepdims=True).T for xb in xs]

    def assign_chunk(b, s, centers2, c2):
        xw = xs[b][s * _NCHUNK:(s + 1) * _NCHUNK]
        mm = jax.lax.dot_general(
            centers2, xw, (((1,), (1,)), ((), ())),
            preferred_element_type=jnp.float32)
        t = (x2rs[b][:, s * _NCHUNK:(s + 1) * _NCHUNK] + c2) - mm
        tmin = jnp.min(t, axis=0, keepdims=True)
        d2min = jnp.maximum(tmin, 0.0)
        mask = t <= d2min
        wi = jnp.where(mask, iota_sub, jnp.float32(_K))
        labels = jnp.min(wi, axis=0, keepdims=True)
        onehot_t = (wi == labels).astype(jnp.float32)
        return xw, d2min, labels, onehot_t

    def step_one(b, centers):
        c2 = jnp.sum(centers * centers, axis=1, keepdims=True)
        centers2 = centers + centers
        sums_aug = jnp.zeros((_K, _D + 1), jnp.float32)
        for s in range(_N // _NCHUNK):
            xw, _, _, onehot_t = assign_chunk(b, s, centers2, c2)
            xw_aug = jnp.concatenate([xw, ones_col], axis=1)
            sums_aug = sums_aug + jnp.dot(
                onehot_t, xw_aug, preferred_element_type=jnp.float32)
        sums = sums_aug[:, :_D]
        counts = sums_aug[:, _D:]
        new_centers = sums / jnp.maximum(counts, 1.0)
        return jnp.where(counts > 0, new_centers, centers)

    def body(_, cs):
        return tuple(step_one(b, cs[b]) for b in range(_BPG))

    centers_t = jax.lax.fori_loop(
        0, _MAX_ITER, body, tuple(c0_ref[b] for b in range(_BPG)))

    for b in range(_BPG):
        centers = centers_t[b]
        c2 = jnp.sum(centers * centers, axis=1, keepdims=True)
        centers2 = centers + centers
        acc = jnp.zeros((), jnp.float32)
        for s in range(_N // _NCHUNK):
            _, d2min, labels, _ = assign_chunk(b, s, centers2, c2)
            labels_ref[b, 0, pl.ds(s * _NCHUNK, _NCHUNK)] = (
                labels.astype(jnp.int32).reshape(_NCHUNK))
            acc = acc + jnp.sum(d2min)
        centers_ref[b] = centers
        inertia_ref[b] = acc.reshape(1, 1)


def kernel(x):
    c0 = _init_centers_like_ref(x)
    labels3, centers, inertia2 = pl.pallas_call(
        _kmeans_body,
        grid=(_BS // _BPG,),
        in_specs=[
            pl.BlockSpec((_BPG, _N, _D), lambda i: (i, 0, 0)),
            pl.BlockSpec((_BPG, _K, _D), lambda i: (i, 0, 0)),
        ],
        out_specs=[
            pl.BlockSpec((_BPG, 1, _N), lambda i: (i, 0, 0)),
            pl.BlockSpec((_BPG, _K, _D), lambda i: (i, 0, 0)),
            pl.BlockSpec((_BPG, 1, 1), lambda i: (i, 0, 0)),
        ],
        out_shape=[
            jax.ShapeDtypeStruct((_BS, 1, _N), jnp.int32),
            jax.ShapeDtypeStruct((_BS, _K, _D), jnp.float32),
            jax.ShapeDtypeStruct((_BS, 1, 1), jnp.float32),
        ],
        compiler_params=pltpu.CompilerParams(
            dimension_semantics=("parallel",)),
    )(x, c0)
    return labels3.reshape(_BS, _N), centers, inertia2.reshape(_BS)


# speedup vs baseline: 1.1982x; 1.0734x over previous
import jax
import jax.numpy as jnp
from jax.experimental import pallas as pl
from jax.experimental.pallas import tpu as pltpu

_BS, _N, _D, _K = 8, 4096, 64, 512
_MAX_ITER = 8
_SEED = 123
_NCHUNK = 4096
_BPG = 2


def _init_centers_like_ref(x):
    key = jax.random.key(_SEED)
    keys = jax.random.split(key, _BS)

    def pick(xi, k):
        idx = jax.random.choice(k, _N, shape=(_K,), replace=False)
        return jnp.take(xi, idx, axis=0)

    return jax.vmap(pick)(x, keys)


def _kmeans_body(x_ref, c0_ref, labels_ref, centers_ref, inertia_ref):
    iota_sub = jax.lax.broadcasted_iota(
        jnp.int32, (_K, _NCHUNK), 0).astype(jnp.float32)
    ones_col = jnp.ones((_NCHUNK, 1), jnp.float32)

    xs = [x_ref[b] for b in range(_BPG)]
    x2rs = [jnp.sum(xb * xb, axis=1, keepdims=True).T for xb in xs]

    def assign_chunk(b, s, centers2, c2):
        xw = xs[b][s * _NCHUNK:(s + 1) * _NCHUNK]
        mm = jax.lax.dot_general(
            centers2, xw, (((1,), (1,)), ((), ())),
            preferred_element_type=jnp.float32)
        t = (x2rs[b][:, s * _NCHUNK:(s + 1) * _NCHUNK] + c2) - mm
        tmin = jnp.min(t, axis=0, keepdims=True)
        d2min = jnp.maximum(tmin, 0.0)
        mask = t <= d2min
        wi = jnp.where(mask, iota_sub, jnp.float32(_K))
        labels = jnp.min(wi, axis=0, keepdims=True)
        onehot_t = (wi == labels).astype(jnp.float32)
        return xw, d2min, labels, onehot_t

    def step_one(b, centers):
        c2 = jnp.sum(centers * centers, axis=1, keepdims=True)
        centers2 = centers + centers
        sums_aug = jnp.zeros((_K, _D + 1), jnp.float32)
        for s in range(_N // _NCHUNK):
            xw, _, _, onehot_t = assign_chunk(b, s, centers2, c2)
            xw_aug = jnp.concatenate([xw, ones_col], axis=1)
            sums_aug = sums_aug + jnp.dot(
                onehot_t, xw_aug, preferred_element_type=jnp.float32)
        sums = sums_aug[:, :_D]
        counts = sums_aug[:, _D:]
        new_centers = sums / jnp.maximum(counts, 1.0)
        return jnp.where(counts > 0, new_centers, centers)

    def body(_, cs):
        return tuple(step_one(b, cs[b]) for b in range(_BPG))

    centers_t = jax.lax.fori_loop(
        0, _MAX_ITER, body, tuple(c0_ref[b] for b in range(_BPG)))

    for b in range(_BPG):
        centers = centers_t[b]
        c2 = jnp.sum(centers * centers, axis=1, keepdims=True)
        centers2 = centers + centers
        acc = jnp.zeros((), jnp.float32)
        for s in range(_N // _NCHUNK):
            _, d2min, labels, _ = assign_chunk(b, s, centers2, c2)
            labels_ref[b, 0, pl.ds(s * _NCHUNK, _NCHUNK)] = (
                labels.astype(jnp.int32).reshape(_NCHUNK))
            acc = acc + jnp.sum(d2min)
        centers_ref[b] = centers
        inertia_ref[b] = acc.reshape(1, 1)


def kernel(x):
    c0 = _init_centers_like_ref(x)
    labels3, centers, inertia2 = pl.pallas_call(
        _kmeans_body,
        grid=(_BS // _BPG,),
        in_specs=[
            pl.BlockSpec((_BPG, _N, _D), lambda i: (i, 0, 0)),
            pl.BlockSpec((_BPG, _K, _D), lambda i: (i, 0, 0)),
        ],
        out_specs=[
            pl.BlockSpec((_BPG, 1, _N), lambda i: (i, 0, 0)),
            pl.BlockSpec((_BPG, _K, _D), lambda i: (i, 0, 0)),
            pl.BlockSpec((_BPG, 1, 1), lambda i: (i, 0, 0)),
        ],
        out_shape=[
            jax.ShapeDtypeStruct((_BS, 1, _N), jnp.int32),
            jax.ShapeDtypeStruct((_BS, _K, _D), jnp.float32),
            jax.ShapeDtypeStruct((_BS, 1, 1), jnp.float32),
        ],
        compiler_params=pltpu.CompilerParams(
            dimension_semantics=("parallel",)),
    )(x, c0)
    return labels3.reshape(_BS, _N), centers, inertia2.reshape(_BS)
